# baseline (speedup 1.0000x reference)
import functools

import jax
import jax.numpy as jnp
import numpy as np
from jax import lax
from jax.experimental import pallas as pl
from jax.experimental.pallas import tpu as pltpu
from jax.experimental.pallas import tpu_sc as plsc

_NEG = 0.01

N = 10000
E = 640000
D = 128
NPAD = 10240
NC, NS = 2, 16
NW = NC * NS
CHUNK = 80
NCHUNK = -(-E // (NW * CHUNK))
NCHUNK += NCHUNK % 2
EPW = NCHUNK * CHUNK
EPAD = NW * EPW - E
BN = 2000
GRID = N // BN


def _lk(v):
    return jnp.where(v >= 0, v, _NEG * v)


def _dot(a, b):
    return jnp.dot(a, b, preferred_element_type=jnp.float32)


def _x1_body(xr, w, b, o):
    o[...] = _lk(_dot(xr[...], w[...]) + b[...])


def _x1_call(x_real, wfc, bfc):
    return pl.pallas_call(
        _x1_body,
        grid=(GRID,),
        in_specs=[
            pl.BlockSpec((BN, 60), lambda i: (i, 0)),
            pl.BlockSpec((60, D), lambda i: (0, 0)),
            pl.BlockSpec((1, D), lambda i: (0, 0)),
        ],
        out_specs=pl.BlockSpec((BN, D), lambda i: (i, 0)),
        out_shape=jax.ShapeDtypeStruct((N, D), jnp.float32),
    )(x_real, wfc, bfc)


def _scatter_partials(x1p, src, dst):
    mesh = plsc.VectorSubcoreMesh(core_axis_name="c", subcore_axis_name="s")
    srcp = jnp.pad(src, (0, EPAD))
    dstp = jnp.pad(dst, (0, EPAD), constant_values=N)
    e4 = jnp.stack([srcp.reshape(NW, NCHUNK, CHUNK),
                    dstp.reshape(NW, NCHUNK, CHUNK)], axis=2)

    @functools.partial(
        pl.kernel,
        out_type=jax.ShapeDtypeStruct((NC * NPAD, D), jnp.float32),
        mesh=mesh,
        scratch_types=[
            pltpu.VMEM((2, CHUNK), jnp.int32),
            pltpu.VMEM((2, CHUNK), jnp.int32),
            pltpu.VMEM((CHUNK, D), jnp.float32),
            pltpu.VMEM((CHUNK, D), jnp.float32),
            pltpu.VMEM_SHARED((NPAD, D), jnp.float32),
            pltpu.SemaphoreType.DMA,
            pltpu.SemaphoreType.DMA,
            pltpu.SemaphoreType.DMA,
            pltpu.SemaphoreType.DMA,
        ],
    )
    def scat(x1_hbm, e_hbm, out_hbm, i0, i1, b0, b1, acc,
             sem0, sem1, semi0, semi1):
        cid = lax.axis_index("c")
        sid = lax.axis_index("s")
        wid = sid * NC + cid

        def zrow(i, _):
            for j in range(D // 16):
                b0[i, pl.ds(j * 16, 16)] = jnp.zeros((16,), jnp.float32)
            return 0

        lax.fori_loop(0, CHUNK, zrow, 0)
        rows_per_tile = NPAD // NS
        rpt = rows_per_tile // CHUNK
        base_r = sid * rows_per_tile
        for j in range(rpt):
            pltpu.sync_copy(b0, acc.at[pl.ds(base_r + j * CHUNK, CHUNK)])
        plsc.subcore_barrier()

        def ldidx(j, ibuf, sem):
            return pltpu.async_copy(e_hbm.at[wid, j], ibuf, sem)

        def gather(ibuf, buf, sem):
            return pltpu.async_copy(x1_hbm.at[ibuf.at[0]], buf, sem)

        def scat_add(ibuf, buf):
            pltpu.sync_copy(buf, acc.at[ibuf.at[1]], add=True)

        ldidx(0, i0, semi0).wait()
        ldidx(1, i1, semi1).wait()
        gather(i0, b0, sem0)

        def body2(k, _):
            j = 2 * k
            g1 = gather(i1, b1, sem1)
            pltpu.make_async_copy(x1_hbm.at[i0.at[0]], b0, sem0).wait()
            scat_add(i0, b0)
            li0 = ldidx(j + 2, i0, semi0)
            g1.wait()
            scat_add(i1, b1)
            li0.wait()
            g2 = gather(i0, b0, sem0)
            li1 = ldidx(j + 3, i1, semi1)
            li1.wait()
            return 0

        lax.fori_loop(0, NCHUNK // 2 - 2, body2, 0)
        j = NCHUNK - 4
        g1 = gather(i1, b1, sem1)
        pltpu.make_async_copy(x1_hbm.at[i0.at[0]], b0, sem0).wait()
        scat_add(i0, b0)
        ldidx(j + 2, i0, semi0).wait()
        g2 = gather(i0, b0, sem0)
        g1.wait()
        scat_add(i1, b1)
        ldidx(j + 3, i1, semi1).wait()
        g2.wait()
        g3 = gather(i1, b1, sem1)
        scat_add(i0, b0)
        g3.wait()
        scat_add(i1, b1)
        plsc.subcore_barrier()

        pltpu.sync_copy(
            acc.at[pl.ds(base_r, rows_per_tile)],
            out_hbm.at[pl.ds(cid * NPAD + base_r, rows_per_tile)])

    return scat(x1p, e4)


def _dense_body(x30, x1, p0, p1, eps1, w1t, b1, w2t, b2,
                a0, a1, bca, mw1a, mb1a, mw2a, mb2a, sa, bsa,
                bmat, bcb, mw1b, mb1b, mw2b, mb2b, sb, bsb,
                cmat, bcc, mw1c, mb1c, mw2c, mb2c, scm, bsc,
                dmat, bcd, o_ref):
    def sig(t):
        return 1.0 / (1.0 + jnp.exp(-t))

    def mlp(t, w1, bb1, w2, bb2):
        return _dot(jnp.maximum(_dot(t, w1) + bb1, 0.0), w2) + bb2

    h = x1[...] * eps1[0, 0] + p0[...] + p1[...]
    h = _lk(_dot(h, w1t[...]) + b1[...])
    h = _lk(_dot(h, w2t[...]) + b2[...])
    x1r = _lk(h)

    v = _lk(_dot(x30[...], a0[...]) + _dot(x1r, a1[...]) + bca[...])

    s = [v[:, i * 64:(i + 1) * 64] for i in range(5)]
    avg = (s[0] + s[1] + s[2] + s[3] + s[4]) * 0.2
    mx = jnp.maximum(jnp.maximum(jnp.maximum(s[0], s[1]),
                                 jnp.maximum(s[2], s[3])), s[4])
    esum = (jnp.exp(s[0] - mx) + jnp.exp(s[1] - mx) + jnp.exp(s[2] - mx)
            + jnp.exp(s[3] - mx) + jnp.exp(s[4] - mx))
    lse = mx + jnp.log(esum)
    att = (mlp(avg, mw1a[...], mb1a[...], mw2a[...], mb2a[...])
           + mlp(mx, mw1a[...], mb1a[...], mw2a[...], mb2a[...])
           + mlp(lse, mw1a[...], mb1a[...], mw2a[...], mb2a[...]))
    g = sig(att)
    s = [si * g for si in s]
    comp = jnp.concatenate(
        [jnp.max(si, axis=1, keepdims=True) for si in s]
        + [jnp.mean(si, axis=1, keepdims=True) for si in s], axis=1)
    sg = sig(_dot(comp, sa[...]) + bsa[...])
    v = jnp.concatenate([s[i] * sg[:, i:i + 1] for i in range(5)], axis=1)

    v = _lk(_dot(v.astype(jnp.bfloat16), bmat[...]) + bcb[...])

    t0, t1 = v[:, :128], v[:, 128:]
    avg = (t0 + t1) * 0.5
    mx = jnp.maximum(t0, t1)
    lse = mx + jnp.log(jnp.exp(t0 - mx) + jnp.exp(t1 - mx))
    att = (mlp(avg, mw1b[...], mb1b[...], mw2b[...], mb2b[...])
           + mlp(mx, mw1b[...], mb1b[...], mw2b[...], mb2b[...])
           + mlp(lse, mw1b[...], mb1b[...], mw2b[...], mb2b[...]))
    g = sig(att)
    t0 = t0 * g
    t1 = t1 * g
    comp = jnp.concatenate(
        [jnp.max(t0, axis=1, keepdims=True), jnp.max(t1, axis=1, keepdims=True),
         jnp.mean(t0, axis=1, keepdims=True), jnp.mean(t1, axis=1, keepdims=True)],
        axis=1)
    sg = sig(_dot(comp, sb[...]) + bsb[...])
    v = jnp.concatenate([t0 * sg[:, 0:1], t1 * sg[:, 1:2]], axis=1)

    v = _lk(_dot(v.astype(jnp.bfloat16), cmat[...]) + bcc[...])

    att = 3.0 * mlp(v, mw1c[...], mb1c[...], mw2c[...], mb2c[...])
    v = v * sig(att)
    comp = jnp.concatenate(
        [jnp.max(v, axis=1, keepdims=True), jnp.mean(v, axis=1, keepdims=True)],
        axis=1)
    sg = sig(_dot(comp, scm[...]) + bsc[...])
    v = v * sg

    o_ref[...] = _lk(_dot(v.astype(jnp.bfloat16), dmat[...]) + bcd[...])


def _dense_call(args):
    blocked = [
        pl.BlockSpec((BN, 30), lambda i: (i, 0)),
        pl.BlockSpec((BN, D), lambda i: (i, 0)),
        pl.BlockSpec((BN, D), lambda i: (i, 0)),
        pl.BlockSpec((BN, D), lambda i: (i, 0)),
    ]
    full = [pl.BlockSpec(a.shape, lambda i: (0, 0)) for a in args[4:]]
    return pl.pallas_call(
        _dense_body,
        grid=(GRID,),
        in_specs=blocked + full,
        out_specs=pl.BlockSpec((BN, 64), lambda i: (i, 0)),
        out_shape=jax.ShapeDtypeStruct((N, 64), jnp.float32),
    )(*args)


def kernel(x, x_real, params, edge_index):
    p = params
    f32 = jnp.float32

    wfc = jnp.pad(p['fc_w'].T, ((0, 0), (0, D - 120)))
    bfc = jnp.pad(p['fc_b'], (0, D - 120))[None]
    x1p = _x1_call(x_real, wfc, bfc)

    part = _scatter_partials(x1p, edge_index[0], edge_index[1])
    p0 = part[:N]
    p1 = part[NPAD:NPAD + N]

    eps1 = (1.0 + p['gin_eps']).reshape(1, 1)
    w1t = jnp.pad(p['gin_w1'].T, ((0, D - 120), (0, 0)))
    b1 = p['gin_b1'][None]
    w2t = p['gin_w2'].T
    b2 = p['gin_b2'][None]

    e1 = (np.arange(15)[:, None, None]
          == 3 * np.arange(5)[None, :, None] + np.arange(2)[None, None, :])
    A = jnp.einsum('hpk,ockw->chwpo', e1.astype(np.float32),
                   p['ca_w']).reshape(60, 320)
    a0, a1 = A[:30], A[30:]
    bca = jnp.tile(p['ca_b'], 5)[None]

    bnsc = 1.0 / np.sqrt(1.0 + 1e-5)
    cba = p['cbam_a']
    mw1a, mb1a = cba['mw1'].T, cba['mb1'][None]
    mw2a, mb2a = cba['mw2'].T, cba['mb2'][None]
    e3 = (np.arange(7)[None, None, :]
          == np.arange(5)[:, None, None] - np.arange(5)[None, :, None] + 3)
    sa = jnp.einsum('hpk,ck->chp', e3.astype(np.float32),
                    cba['sw'][0, :, :, 3]).reshape(10, 5) * (cba['bn_w'][0] * bnsc)
    bsa = jnp.tile(cba['bn_b'], 5)[None]

    e2 = (np.arange(5)[:, None, None]
          == 2 * np.arange(2)[None, :, None] + np.arange(2)[None, None, :])
    B = jnp.einsum('pqk,ock->pcqo', e2.astype(np.float32),
                   p['cb_w'][:, :, :, 0]).reshape(320, 256).astype(jnp.bfloat16)
    bcb = jnp.tile(p['cb_b'], 2)[None]

    cbb = p['cbam_b']
    mw1b, mb1b = cbb['mw1'].T, cbb['mb1'][None]
    mw2b, mb2b = cbb['mw2'].T, cbb['mb2'][None]
    e4 = (np.arange(7)[None, None, :]
          == np.arange(2)[:, None, None] - np.arange(2)[None, :, None] + 3)
    sb = jnp.einsum('hpk,ck->chp', e4.astype(np.float32),
                    cbb['sw'][0, :, :, 3]).reshape(4, 2) * (cbb['bn_w'][0] * bnsc)
    bsb = jnp.tile(cbb['bn_b'], 2)[None]

    C = jnp.transpose(p['cc_w'][:, :, :, 0],
                      (2, 1, 0)).reshape(256, 256).astype(jnp.bfloat16)
    bcc = p['cc_b'][None]

    cbc = p['cbam_c']
    mw1c, mb1c = cbc['mw1'].T, cbc['mb1'][None]
    mw2c, mb2c = cbc['mw2'].T, cbc['mb2'][None]
    scm = jnp.stack([cbc['sw'][0, 0, 3, 3],
                     cbc['sw'][0, 1, 3, 3]])[:, None] * (cbc['bn_w'][0] * bnsc)
    bsc = cbc['bn_b'][None]

    dmat = p['cd_w'][:, :, 0, 0].T.astype(jnp.bfloat16)
    bcd = p['cd_b'][None]

    x30 = x.reshape(N, 30)
    out = _dense_call([
        x30, x1p, p0, p1, eps1, w1t, b1, w2t, b2,
        a0, a1, bca, mw1a, mb1a, mw2a, mb2a, sa, bsa,
        B, bcb, mw1b, mb1b, mw2b, mb2b, sb, bsb,
        C, bcc, mw1c, mb1c, mw2c, mb2c, scm, bsc,
        dmat, bcd,
    ])
    return out.reshape(N, 64, 1, 1)


# speedup vs baseline: 1.0022x; 1.0022x over previous
import functools

import jax
import jax.numpy as jnp
import numpy as np
from jax import lax
from jax.experimental import pallas as pl
from jax.experimental.pallas import tpu as pltpu
from jax.experimental.pallas import tpu_sc as plsc

_NEG = 0.01

N = 10000
E = 640000
D = 128
NPAD = 10240
NC, NS = 2, 16
NW = NC * NS
CHUNK = 80
NCHUNK = -(-E // (NW * CHUNK))
NCHUNK += NCHUNK % 2
EPW = NCHUNK * CHUNK
EPAD = NW * EPW - E
BN = 2000
GRID = N // BN


def _lk(v):
    return jnp.where(v >= 0, v, _NEG * v)


def _dot(a, b):
    return jnp.dot(a, b, preferred_element_type=jnp.float32)


def _x1_body(xr, w, b, o):
    o[...] = _lk(_dot(xr[...], w[...]) + b[...])


def _x1_call(x_real, wfc, bfc):
    return pl.pallas_call(
        _x1_body,
        grid=(GRID,),
        in_specs=[
            pl.BlockSpec((BN, 60), lambda i: (i, 0)),
            pl.BlockSpec((60, D), lambda i: (0, 0)),
            pl.BlockSpec((1, D), lambda i: (0, 0)),
        ],
        out_specs=pl.BlockSpec((BN, D), lambda i: (i, 0)),
        out_shape=jax.ShapeDtypeStruct((N, D), jnp.float32),
    )(x_real, wfc, bfc)


def _scatter_partials(x1p, src, dst):
    mesh = plsc.VectorSubcoreMesh(core_axis_name="c", subcore_axis_name="s")
    srcp = jnp.pad(src, (0, EPAD))
    dstp = jnp.pad(dst, (0, EPAD), constant_values=N)
    e4 = jnp.stack([srcp.reshape(NW, NCHUNK, CHUNK),
                    dstp.reshape(NW, NCHUNK, CHUNK)], axis=2)

    @functools.partial(
        pl.kernel,
        out_type=jax.ShapeDtypeStruct((NC * NPAD, D), jnp.float32),
        mesh=mesh,
        scratch_types=[
            pltpu.VMEM((2, CHUNK), jnp.int32),
            pltpu.VMEM((2, CHUNK), jnp.int32),
            pltpu.VMEM((CHUNK, D), jnp.float32),
            pltpu.VMEM((CHUNK, D), jnp.float32),
            pltpu.VMEM_SHARED((NPAD, D), jnp.float32),
            pltpu.SemaphoreType.DMA,
            pltpu.SemaphoreType.DMA,
            pltpu.SemaphoreType.DMA,
            pltpu.SemaphoreType.DMA,
        ],
    )
    def scat(x1_hbm, e_hbm, out_hbm, i0, i1, b0, b1, acc,
             sem0, sem1, semi0, semi1):
        cid = lax.axis_index("c")
        sid = lax.axis_index("s")
        wid = sid * NC + cid

        def zrow(i, _):
            for j in range(D // 16):
                b0[i, pl.ds(j * 16, 16)] = jnp.zeros((16,), jnp.float32)
            return 0

        lax.fori_loop(0, CHUNK, zrow, 0)
        rows_per_tile = NPAD // NS
        rpt = rows_per_tile // CHUNK
        base_r = sid * rows_per_tile
        for j in range(rpt):
            pltpu.sync_copy(b0, acc.at[pl.ds(base_r + j * CHUNK, CHUNK)])
        plsc.subcore_barrier()

        def ldidx(j, ibuf, sem):
            return pltpu.async_copy(e_hbm.at[wid, j], ibuf, sem)

        def gather(ibuf, buf, sem):
            return pltpu.async_copy(x1_hbm.at[ibuf.at[0]], buf, sem)

        def scat_add(ibuf, buf):
            pltpu.sync_copy(buf, acc.at[ibuf.at[1]], add=True)

        ldidx(0, i0, semi0).wait()
        ldidx(1, i1, semi1).wait()
        gather(i0, b0, sem0)

        def body2(k, _):
            j = 2 * k
            g1 = gather(i1, b1, sem1)
            pltpu.make_async_copy(x1_hbm.at[i0.at[0]], b0, sem0).wait()
            scat_add(i0, b0)
            li0 = ldidx(j + 2, i0, semi0)
            li0.wait()
            g2 = gather(i0, b0, sem0)
            g1.wait()
            scat_add(i1, b1)
            li1 = ldidx(j + 3, i1, semi1)
            li1.wait()
            return 0

        lax.fori_loop(0, NCHUNK // 2 - 2, body2, 0)
        j = NCHUNK - 4
        g1 = gather(i1, b1, sem1)
        pltpu.make_async_copy(x1_hbm.at[i0.at[0]], b0, sem0).wait()
        scat_add(i0, b0)
        ldidx(j + 2, i0, semi0).wait()
        g2 = gather(i0, b0, sem0)
        g1.wait()
        scat_add(i1, b1)
        ldidx(j + 3, i1, semi1).wait()
        g2.wait()
        g3 = gather(i1, b1, sem1)
        scat_add(i0, b0)
        g3.wait()
        scat_add(i1, b1)
        plsc.subcore_barrier()

        pltpu.sync_copy(
            acc.at[pl.ds(base_r, rows_per_tile)],
            out_hbm.at[pl.ds(cid * NPAD + base_r, rows_per_tile)])

    return scat(x1p, e4)


def _dense_body(x30, x1, p0, p1, eps1, w1t, b1, w2t, b2,
                a0, a1, bca, mw1a, mb1a, mw2a, mb2a, sa, bsa,
                bmat, bcb, mw1b, mb1b, mw2b, mb2b, sb, bsb,
                cmat, bcc, mw1c, mb1c, mw2c, mb2c, scm, bsc,
                dmat, bcd, o_ref):
    def sig(t):
        return 1.0 / (1.0 + jnp.exp(-t))

    def mlp(t, w1, bb1, w2, bb2):
        return _dot(jnp.maximum(_dot(t, w1) + bb1, 0.0), w2) + bb2

    h = x1[...] * eps1[0, 0] + p0[...] + p1[...]
    h = _lk(_dot(h, w1t[...]) + b1[...])
    h = _lk(_dot(h, w2t[...]) + b2[...])
    x1r = _lk(h)

    v = _lk(_dot(x30[...], a0[...]) + _dot(x1r, a1[...]) + bca[...])

    s = [v[:, i * 64:(i + 1) * 64] for i in range(5)]
    avg = (s[0] + s[1] + s[2] + s[3] + s[4]) * 0.2
    mx = jnp.maximum(jnp.maximum(jnp.maximum(s[0], s[1]),
                                 jnp.maximum(s[2], s[3])), s[4])
    esum = (jnp.exp(s[0] - mx) + jnp.exp(s[1] - mx) + jnp.exp(s[2] - mx)
            + jnp.exp(s[3] - mx) + jnp.exp(s[4] - mx))
    lse = mx + jnp.log(esum)
    att = (mlp(avg, mw1a[...], mb1a[...], mw2a[...], mb2a[...])
           + mlp(mx, mw1a[...], mb1a[...], mw2a[...], mb2a[...])
           + mlp(lse, mw1a[...], mb1a[...], mw2a[...], mb2a[...]))
    g = sig(att)
    s = [si * g for si in s]
    comp = jnp.concatenate(
        [jnp.max(si, axis=1, keepdims=True) for si in s]
        + [jnp.mean(si, axis=1, keepdims=True) for si in s], axis=1)
    sg = sig(_dot(comp, sa[...]) + bsa[...])
    v = jnp.concatenate([s[i] * sg[:, i:i + 1] for i in range(5)], axis=1)

    v = _lk(_dot(v.astype(jnp.bfloat16), bmat[...]) + bcb[...])

    t0, t1 = v[:, :128], v[:, 128:]
    avg = (t0 + t1) * 0.5
    mx = jnp.maximum(t0, t1)
    lse = mx + jnp.log(jnp.exp(t0 - mx) + jnp.exp(t1 - mx))
    att = (mlp(avg, mw1b[...], mb1b[...], mw2b[...], mb2b[...])
           + mlp(mx, mw1b[...], mb1b[...], mw2b[...], mb2b[...])
           + mlp(lse, mw1b[...], mb1b[...], mw2b[...], mb2b[...]))
    g = sig(att)
    t0 = t0 * g
    t1 = t1 * g
    comp = jnp.concatenate(
        [jnp.max(t0, axis=1, keepdims=True), jnp.max(t1, axis=1, keepdims=True),
         jnp.mean(t0, axis=1, keepdims=True), jnp.mean(t1, axis=1, keepdims=True)],
        axis=1)
    sg = sig(_dot(comp, sb[...]) + bsb[...])
    v = jnp.concatenate([t0 * sg[:, 0:1], t1 * sg[:, 1:2]], axis=1)

    v = _lk(_dot(v.astype(jnp.bfloat16), cmat[...]) + bcc[...])

    att = 3.0 * mlp(v, mw1c[...], mb1c[...], mw2c[...], mb2c[...])
    v = v * sig(att)
    comp = jnp.concatenate(
        [jnp.max(v, axis=1, keepdims=True), jnp.mean(v, axis=1, keepdims=True)],
        axis=1)
    sg = sig(_dot(comp, scm[...]) + bsc[...])
    v = v * sg

    o_ref[...] = _lk(_dot(v.astype(jnp.bfloat16), dmat[...]) + bcd[...])


def _dense_call(args):
    blocked = [
        pl.BlockSpec((BN, 30), lambda i: (i, 0)),
        pl.BlockSpec((BN, D), lambda i: (i, 0)),
        pl.BlockSpec((BN, D), lambda i: (i, 0)),
        pl.BlockSpec((BN, D), lambda i: (i, 0)),
    ]
    full = [pl.BlockSpec(a.shape, lambda i: (0, 0)) for a in args[4:]]
    return pl.pallas_call(
        _dense_body,
        grid=(GRID,),
        in_specs=blocked + full,
        out_specs=pl.BlockSpec((BN, 64), lambda i: (i, 0)),
        out_shape=jax.ShapeDtypeStruct((N, 64), jnp.float32),
    )(*args)


def kernel(x, x_real, params, edge_index):
    p = params
    f32 = jnp.float32

    wfc = jnp.pad(p['fc_w'].T, ((0, 0), (0, D - 120)))
    bfc = jnp.pad(p['fc_b'], (0, D - 120))[None]
    x1p = _x1_call(x_real, wfc, bfc)

    part = _scatter_partials(x1p, edge_index[0], edge_index[1])
    p0 = part[:N]
    p1 = part[NPAD:NPAD + N]

    eps1 = (1.0 + p['gin_eps']).reshape(1, 1)
    w1t = jnp.pad(p['gin_w1'].T, ((0, D - 120), (0, 0)))
    b1 = p['gin_b1'][None]
    w2t = p['gin_w2'].T
    b2 = p['gin_b2'][None]

    e1 = (np.arange(15)[:, None, None]
          == 3 * np.arange(5)[None, :, None] + np.arange(2)[None, None, :])
    A = jnp.einsum('hpk,ockw->chwpo', e1.astype(np.float32),
                   p['ca_w']).reshape(60, 320)
    a0, a1 = A[:30], A[30:]
    bca = jnp.tile(p['ca_b'], 5)[None]

    bnsc = 1.0 / np.sqrt(1.0 + 1e-5)
    cba = p['cbam_a']
    mw1a, mb1a = cba['mw1'].T, cba['mb1'][None]
    mw2a, mb2a = cba['mw2'].T, cba['mb2'][None]
    e3 = (np.arange(7)[None, None, :]
          == np.arange(5)[:, None, None] - np.arange(5)[None, :, None] + 3)
    sa = jnp.einsum('hpk,ck->chp', e3.astype(np.float32),
                    cba['sw'][0, :, :, 3]).reshape(10, 5) * (cba['bn_w'][0] * bnsc)
    bsa = jnp.tile(cba['bn_b'], 5)[None]

    e2 = (np.arange(5)[:, None, None]
          == 2 * np.arange(2)[None, :, None] + np.arange(2)[None, None, :])
    B = jnp.einsum('pqk,ock->pcqo', e2.astype(np.float32),
                   p['cb_w'][:, :, :, 0]).reshape(320, 256).astype(jnp.bfloat16)
    bcb = jnp.tile(p['cb_b'], 2)[None]

    cbb = p['cbam_b']
    mw1b, mb1b = cbb['mw1'].T, cbb['mb1'][None]
    mw2b, mb2b = cbb['mw2'].T, cbb['mb2'][None]
    e4 = (np.arange(7)[None, None, :]
          == np.arange(2)[:, None, None] - np.arange(2)[None, :, None] + 3)
    sb = jnp.einsum('hpk,ck->chp', e4.astype(np.float32),
                    cbb['sw'][0, :, :, 3]).reshape(4, 2) * (cbb['bn_w'][0] * bnsc)
    bsb = jnp.tile(cbb['bn_b'], 2)[None]

    C = jnp.transpose(p['cc_w'][:, :, :, 0],
                      (2, 1, 0)).reshape(256, 256).astype(jnp.bfloat16)
    bcc = p['cc_b'][None]

    cbc = p['cbam_c']
    mw1c, mb1c = cbc['mw1'].T, cbc['mb1'][None]
    mw2c, mb2c = cbc['mw2'].T, cbc['mb2'][None]
    scm = jnp.stack([cbc['sw'][0, 0, 3, 3],
                     cbc['sw'][0, 1, 3, 3]])[:, None] * (cbc['bn_w'][0] * bnsc)
    bsc = cbc['bn_b'][None]

    dmat = p['cd_w'][:, :, 0, 0].T.astype(jnp.bfloat16)
    bcd = p['cd_b'][None]

    x30 = x.reshape(N, 30)
    out = _dense_call([
        x30, x1p, p0, p1, eps1, w1t, b1, w2t, b2,
        a0, a1, bca, mw1a, mb1a, mw2a, mb2a, sa, bsa,
        B, bcb, mw1b, mb1b, mw2b, mb2b, sb, bsb,
        C, bcc, mw1c, mb1c, mw2c, mb2c, scm, bsc,
        dmat, bcd,
    ])
    return out.reshape(N, 64, 1, 1)
